# baseline (speedup 1.0000x reference)
import jax
import jax.numpy as jnp
from jax.experimental import pallas as pl
from jax.experimental.pallas import tpu as pltpu

_BR = 200


def _gcn_kernel(adj_ref, x_ref, w1_ref, b1_ref, w2_ref, b2_ref, o_ref,
                a_scr, g_scr):
    p = pl.program_id(0)
    i = pl.program_id(1)

    @pl.when(jnp.logical_and(p == 0, i == 0))
    def _():
        a_scr[...] = jnp.dot(x_ref[...], w1_ref[...],
                             preferred_element_type=jnp.float32)

    @pl.when(p == 0)
    def _():
        h = jnp.dot(adj_ref[...], a_scr[...],
                    preferred_element_type=jnp.float32)
        h = jnp.maximum(h + b1_ref[...], 0.0)
        g_scr[pl.ds(i * _BR, _BR), :] = jnp.dot(
            h, w2_ref[...], preferred_element_type=jnp.float32)

    @pl.when(p == 1)
    def _():
        z = jnp.dot(adj_ref[...], g_scr[...],
                    preferred_element_type=jnp.float32)
        z = jnp.maximum(z + b2_ref[...], 0.0)
        m = jnp.max(z, axis=1, keepdims=True)
        s = z - m
        lse = jnp.log(jnp.sum(jnp.exp(s), axis=1, keepdims=True))
        o_ref[...] = s - lse


def kernel(x, adj, W1, b1, W2, b2):
    n, d_in = x.shape
    hid = W1.shape[1]
    classes = W2.shape[1]
    b1r = b1.reshape(1, hid)
    b2r = b2.reshape(1, classes)

    nb = n // _BR
    return pl.pallas_call(
        _gcn_kernel,
        grid=(2, nb),
        in_specs=[
            pl.BlockSpec((_BR, n), lambda p, i: (i + p * (nb - 1 - 2 * i), 0)),
            pl.BlockSpec((n, d_in), lambda p, i: (0, 0)),
            pl.BlockSpec((d_in, hid), lambda p, i: (0, 0)),
            pl.BlockSpec((1, hid), lambda p, i: (0, 0)),
            pl.BlockSpec((hid, classes), lambda p, i: (0, 0)),
            pl.BlockSpec((1, classes), lambda p, i: (0, 0)),
        ],
        out_specs=pl.BlockSpec((_BR, classes),
                               lambda p, i: (p * (nb - 1 - i), 0)),
        out_shape=jax.ShapeDtypeStruct((n, classes), jnp.float32),
        scratch_shapes=[
            pltpu.VMEM((n, hid), jnp.float32),
            pltpu.VMEM((n, classes), jnp.float32),
        ],
    )(adj, x, W1, b1r, W2, b2r)
